# baseline (speedup 1.0000x reference)
import functools

import jax
import jax.numpy as jnp
from jax import lax
from jax.experimental import pallas as pl
from jax.experimental.pallas import tpu as pltpu
from jax.experimental.pallas import tpu_sc as plsc

_H = 1024
_W = 1024
_STRIDE = 126
_GRID = 9
_C = 16
_NR = 16
_ITEMS_PER_W = _C * (_H // _NR) // 32
_STEPS = _ITEMS_PER_W // 2
_BUFP = 144


def _sc_body(x, out, bufs, rows, in_sem, in2_sem, out_sem):
    cid = lax.axis_index("c")
    sid = lax.axis_index("s")
    wid = cid * 16 + sid

    def _coords(item):
        a = wid * _ITEMS_PER_W + item
        c = a // (_H // _NR)
        q = a % (_H // _NR)
        h0 = q * _NR
        i0 = h0 // _STRIDE
        i1 = (h0 + _NR - 1) // _STRIDE
        base = jnp.minimum(h0 - i0 * _STRIDE, 128 - _NR)
        return c, q, h0, i0, i1, base

    def _in_copy(item, b):
        c, q, h0, i0, i1, base = _coords(item)
        return pltpu.make_async_copy(
            x.at[pl.ds(i0 * _GRID, _GRID), c, pl.ds(base, _NR), :],
            bufs.at[b, 0, :, :, pl.ds(0, 128)],
            in_sem.at[b],
        )

    def _in2_copy(item, b):
        c, q, h0, i0, i1, base = _coords(item)
        return pltpu.make_async_copy(
            x.at[pl.ds(i1 * _GRID, _GRID), c, pl.ds(0, _NR), :],
            bufs.at[b, 1, :, :, pl.ds(0, 128)],
            in2_sem.at[b],
        )

    def _straddles(item):
        _, _, _, i0, i1, _ = _coords(item)
        return i1 > i0

    def _out_copy(item, b, half):
        c, q, h0, i0, i1, base = _coords(item)
        return pltpu.make_async_copy(
            rows.at[b, half],
            out.at[c, 2 * q + half, :, :, :],
            out_sem.at[b],
        )

    def _start_in(item, b):
        _in_copy(item, b).start()

        @pl.when(_straddles(item))
        def _():
            _in2_copy(item, b).start()

    def _wait_in(item, b):
        _in_copy(item, b).wait()

        @pl.when(_straddles(item))
        def _():
            _in2_copy(item, b).wait()

    def _build_row(b, slot, rw, half, rr):
        iota = lax.iota(jnp.int32, 16)
        for v in range(64):
            w0 = 16 * v
            j = w0 // _STRIDE
            t, off = divmod(w0, 128)
            a = bufs[b, slot, j, rw, pl.ds(w0 - j * _STRIDE, 16)]
            bound = (j + 1) * _STRIDE
            if j < 8 and w0 + 16 > bound:
                d = bound - w0
                nxt = bufs[b, slot, j + 1, rw, pl.ds(0, 16)]
                idx = jnp.maximum(iota - d, 0)
                shifted = lax.gather(
                    nxt,
                    idx[:, None],
                    lax.GatherDimensionNumbers(
                        offset_dims=(),
                        collapsed_slice_dims=(0,),
                        start_index_map=(0,),
                    ),
                    (1,),
                    mode=lax.GatherScatterMode.PROMISE_IN_BOUNDS,
                )
                a = jnp.where(iota < d, a, shifted)
            rows[b, half, t, rr, pl.ds(off, 16)] = a

    def _assemble_rows(item, b):
        _, _, h0, i0, i1, base = _coords(item)

        @pl.when(i1 == i0)
        def _pure():
            for half in range(2):

                @plsc.parallel_loop(0, 8, 1)
                def _row(r8):
                    _build_row(b, 0, 8 * half + r8, half, r8)

        @pl.when(i1 > i0)
        def _strad():
            @plsc.parallel_loop(0, _NR, 1)
            def _row(r):
                h = h0 + r
                ih = h // _STRIDE
                slot = ih - i0
                rw = h - ih * _STRIDE - jnp.where(slot == 0, base, 0)
                _build_row(b, slot, rw, r // 8, r % 8)

    _start_in(0, 0)
    _start_in(1, 1)

    def _pipe(s, carry):
        for b in range(2):
            item = 2 * s + b

            @pl.when(item >= 2)
            def _wait_out():
                _out_copy(item - 2, b, 0).wait()
                _out_copy(item - 2, b, 1).wait()

            _wait_in(item, b)
            _assemble_rows(item, b)
            _out_copy(item, b, 0).start()
            _out_copy(item, b, 1).start()

            @pl.when(item + 2 < _ITEMS_PER_W)
            def _next_in():
                _start_in(item + 2, b)

        return carry

    lax.fori_loop(0, _STEPS, _pipe, 0)
    for b in range(2):
        _out_copy(_ITEMS_PER_W - 2 + b, b, 0).wait()
        _out_copy(_ITEMS_PER_W - 2 + b, b, 1).wait()


_sc_kernel = functools.partial(
    pl.kernel,
    out_type=jax.ShapeDtypeStruct((_C, _H // 8, 8, 8, 128), jnp.float32),
    mesh=plsc.VectorSubcoreMesh(core_axis_name="c", subcore_axis_name="s"),
    scratch_types=[
        pltpu.VMEM((2, 2, _GRID, _NR, _BUFP), jnp.float32),
        pltpu.VMEM((2, 2, 8, 8, 128), jnp.float32),
        pltpu.SemaphoreType.DMA((2,)),
        pltpu.SemaphoreType.DMA((2,)),
        pltpu.SemaphoreType.DMA((2,)),
    ],
    compiler_params=pltpu.CompilerParams(use_tc_tiling_on_sc=False),
)(_sc_body)


def kernel(x):
    o = _sc_kernel(x)
    o = o.transpose(0, 1, 3, 2, 4)
    return o.reshape(_C, _H, _W)


# speedup vs baseline: 1.3928x; 1.3928x over previous
import functools

import jax
import jax.numpy as jnp
from jax import lax
from jax.experimental import pallas as pl
from jax.experimental.pallas import tpu as pltpu
from jax.experimental.pallas import tpu_sc as plsc

_H = 1024
_W = 1024
_STRIDE = 126
_GRID = 9
_C = 16
_NR = 16
_ITEMS_PER_W = _C * (_H // _NR) // 32
_STEPS = _ITEMS_PER_W // 2
_BUFP = 144


def _sc_body(x, out, bufs, rows, in_sem, in2_sem, out_sem):
    cid = lax.axis_index("c")
    sid = lax.axis_index("s")
    wid = cid * 16 + sid

    def _coords(item):
        a = wid * _ITEMS_PER_W + item
        c = a // (_H // _NR)
        q = a % (_H // _NR)
        h0 = q * _NR
        i0 = h0 // _STRIDE
        i1 = (h0 + _NR - 1) // _STRIDE
        return c, q, h0, i0, i1

    def _in_copy(item, b):
        c, q, h0, i0, i1 = _coords(item)
        return pltpu.make_async_copy(
            x.at[pl.ds(i0 * _GRID, _GRID), c, pl.ds(h0 - i0 * _STRIDE, _NR), :],
            bufs.at[b, :, pl.ds(0, _NR), pl.ds(0, 128)],
            in_sem.at[b],
        )

    def _in2_copy(item, b):
        c, q, h0, i0, i1 = _coords(item)
        n0 = i1 * _STRIDE - h0
        return pltpu.make_async_copy(
            x.at[pl.ds(i1 * _GRID, _GRID), c, pl.ds(0, _NR), :],
            bufs.at[b, :, pl.ds(n0, _NR), pl.ds(0, 128)],
            in2_sem.at[b],
        )

    def _straddles(item):
        _, _, _, i0, i1 = _coords(item)
        return i1 > i0

    def _out_copy(item, b, half):
        c, q, h0, i0, i1 = _coords(item)
        return pltpu.make_async_copy(
            rows.at[b, :, pl.ds(8 * half, 8), :],
            out.at[c, 2 * q + half, :, :, :],
            out_sem.at[b],
        )

    def _wait_in(item, b):
        _in_copy(item, b).wait()

        @pl.when(_straddles(item))
        def _():
            _in2_copy(item, b).start()
            _in2_copy(item, b).wait()

    def _assemble_rows(b):
        iota = lax.iota(jnp.int32, 16)

        @plsc.parallel_loop(0, _NR, 1)
        def _row(r):
            for v in range(64):
                w0 = 16 * v
                j = w0 // _STRIDE
                t, off = divmod(w0, 128)
                a = bufs[b, j, r, pl.ds(w0 - j * _STRIDE, 16)]
                bound = (j + 1) * _STRIDE
                if j < 8 and w0 + 16 > bound:
                    d = bound - w0
                    nxt = bufs[b, j + 1, r, pl.ds(0, 16)]
                    idx = jnp.maximum(iota - d, 0)
                    shifted = lax.gather(
                        nxt,
                        idx[:, None],
                        lax.GatherDimensionNumbers(
                            offset_dims=(),
                            collapsed_slice_dims=(0,),
                            start_index_map=(0,),
                        ),
                        (1,),
                        mode=lax.GatherScatterMode.PROMISE_IN_BOUNDS,
                    )
                    a = jnp.where(iota < d, a, shifted)
                rows[b, t, r, pl.ds(off, 16)] = a

    _in_copy(0, 0).start()
    _in_copy(1, 1).start()

    def _pipe(s, carry):
        for b in range(2):
            item = 2 * s + b

            @pl.when(item >= 2)
            def _wait_out():
                _out_copy(item - 2, b, 0).wait()
                _out_copy(item - 2, b, 1).wait()

            _wait_in(item, b)
            _assemble_rows(b)
            _out_copy(item, b, 0).start()
            _out_copy(item, b, 1).start()

            @pl.when(item + 2 < _ITEMS_PER_W)
            def _next_in():
                _in_copy(item + 2, b).start()

        return carry

    lax.fori_loop(0, _STEPS, _pipe, 0)
    for b in range(2):
        _out_copy(_ITEMS_PER_W - 2 + b, b, 0).wait()
        _out_copy(_ITEMS_PER_W - 2 + b, b, 1).wait()


_sc_kernel = functools.partial(
    pl.kernel,
    out_type=jax.ShapeDtypeStruct((_C, _H // 8, 8, 8, 128), jnp.float32),
    mesh=plsc.VectorSubcoreMesh(core_axis_name="c", subcore_axis_name="s"),
    scratch_types=[
        pltpu.VMEM((2, _GRID, 2 * _NR, _BUFP), jnp.float32),
        pltpu.VMEM((2, 8, _NR, 128), jnp.float32),
        pltpu.SemaphoreType.DMA((2,)),
        pltpu.SemaphoreType.DMA((2,)),
        pltpu.SemaphoreType.DMA((2,)),
    ],
    compiler_params=pltpu.CompilerParams(use_tc_tiling_on_sc=False),
)(_sc_body)


def kernel(x):
    o = _sc_kernel(x)
    o = o.transpose(0, 1, 3, 2, 4)
    return o.reshape(_C, _H, _W)


# speedup vs baseline: 1.5915x; 1.1427x over previous
import functools

import jax
import jax.numpy as jnp
from jax import lax
from jax.experimental import pallas as pl
from jax.experimental.pallas import tpu as pltpu
from jax.experimental.pallas import tpu_sc as plsc

_H = 1024
_W = 1024
_STRIDE = 126
_GRID = 9
_C = 16
_NR = 21
_CHUNKS = _STRIDE // _NR
_ITEMS_PER_W = _C * 8 * _CHUNKS // 32
_STEPS = _ITEMS_PER_W // 2
_BUFP = 144


def _sc_body(x, out, bufs, rows, in_sem, out_sem):
    cid = lax.axis_index("c")
    sid = lax.axis_index("s")
    wid = cid * 16 + sid

    def _coords(item):
        a = wid * _ITEMS_PER_W + item
        c = a // (8 * _CHUNKS)
        rem = a % (8 * _CHUNKS)
        i = rem // _CHUNKS
        r0 = (rem % _CHUNKS) * _NR
        return c, i, r0

    def _in_copy(item, b):
        c, i, r0 = _coords(item)
        return pltpu.make_async_copy(
            x.at[pl.ds(i * _GRID, _GRID), c, pl.ds(r0, _NR), :],
            bufs.at[b, :, :, pl.ds(0, 128)],
            in_sem.at[b],
        )

    def _row_out_copy(item, b, r):
        c, i, r0 = _coords(item)
        h = i * _STRIDE + r0 + r
        return pltpu.make_async_copy(
            rows.at[b, r],
            out.at[c, h // 8, :, h % 8, :],
            out_sem.at[b],
        )

    def _assemble_rows(b, nrows):
        iota = lax.iota(jnp.int32, 16)

        @plsc.parallel_loop(0, nrows, 1)
        def _row(r):
            for v in range(64):
                w0 = 16 * v
                j = w0 // _STRIDE
                t, off = divmod(w0, 128)
                a = bufs[b, j, r, pl.ds(w0 - j * _STRIDE, 16)]
                bound = (j + 1) * _STRIDE
                if j < 8 and w0 + 16 > bound:
                    d = bound - w0
                    nxt = bufs[b, j + 1, r, pl.ds(0, 16)]
                    idx = jnp.maximum(iota - d, 0)
                    shifted = lax.gather(
                        nxt,
                        idx[:, None],
                        lax.GatherDimensionNumbers(
                            offset_dims=(),
                            collapsed_slice_dims=(0,),
                            start_index_map=(0,),
                        ),
                        (1,),
                        mode=lax.GatherScatterMode.PROMISE_IN_BOUNDS,
                    )
                    a = jnp.where(iota < d, a, shifted)
                rows[b, r, t, pl.ds(off, 16)] = a

    def _drain_out(b, nrows):
        pltpu.make_async_copy(
            x.at[pl.ds(0, nrows), 0, pl.ds(0, 8), :],
            rows.at[b, pl.ds(0, nrows)],
            out_sem.at[b],
        ).wait()

    _in_copy(0, 0).start()
    _in_copy(1, 1).start()

    def _pipe(t, carry):
        for b in range(2):
            item = 2 * t + b

            @pl.when(item >= 2)
            def _wait_out():
                _drain_out(b, _NR)

            _in_copy(item, b).wait()
            _assemble_rows(b, _NR)
            for r in range(_NR):
                _row_out_copy(item, b, r).start()

            @pl.when(item + 2 < _ITEMS_PER_W)
            def _next_in():
                _in_copy(item + 2, b).start()

        return carry

    lax.fori_loop(0, _STEPS, _pipe, 0)
    _drain_out(0, _NR)
    _drain_out(1, _NR)

    @pl.when(wid < _C)
    def _():
        c = wid
        pltpu.sync_copy(
            x.at[pl.ds(8 * _GRID, _GRID), c, pl.ds(0, 16), :],
            bufs.at[0, :, pl.ds(0, 16), pl.ds(0, 128)],
        )
        _assemble_rows(0, 16)
        for r in range(16):
            h = 8 * _STRIDE + r
            pltpu.make_async_copy(
                rows.at[0, r],
                out.at[c, h // 8, :, h % 8, :],
                out_sem.at[0],
            ).start()
        _drain_out(0, 16)


_sc_kernel = functools.partial(
    pl.kernel,
    out_type=jax.ShapeDtypeStruct((_C, _H // 8, 8, 8, 128), jnp.float32),
    mesh=plsc.VectorSubcoreMesh(core_axis_name="c", subcore_axis_name="s"),
    scratch_types=[
        pltpu.VMEM((2, _GRID, _NR, _BUFP), jnp.float32),
        pltpu.VMEM((2, _NR, 8, 128), jnp.float32),
        pltpu.SemaphoreType.DMA((2,)),
        pltpu.SemaphoreType.DMA((2,)),
    ],
    compiler_params=pltpu.CompilerParams(use_tc_tiling_on_sc=False),
)(_sc_body)


def kernel(x):
    o = _sc_kernel(x)
    o = o.transpose(0, 1, 3, 2, 4)
    return o.reshape(_C, _H, _W)


# speedup vs baseline: 1.6420x; 1.0318x over previous
import functools

import jax
import jax.numpy as jnp
from jax import lax
from jax.experimental import pallas as pl
from jax.experimental.pallas import tpu as pltpu
from jax.experimental.pallas import tpu_sc as plsc

_H = 1024
_W = 1024
_STRIDE = 126
_GRID = 9
_C = 16
_NR = 21
_CHUNKS = _STRIDE // _NR
_ITEMS_PER_W = _C * 8 * _CHUNKS // 32
_STEPS = _ITEMS_PER_W // 2
_BUFP = 144


def _sc_body(x, out, bufs, rows, in_sem, out_sem):
    cid = lax.axis_index("c")
    sid = lax.axis_index("s")
    wid = cid * 16 + sid

    def _coords(item):
        a = wid * _ITEMS_PER_W + item
        c = a // (8 * _CHUNKS)
        rem = a % (8 * _CHUNKS)
        i = rem // _CHUNKS
        r0 = (rem % _CHUNKS) * _NR
        return c, i, r0

    def _in_copy(item, b):
        c, i, r0 = _coords(item)
        return pltpu.make_async_copy(
            x.at[pl.ds(i * _GRID, _GRID), c, pl.ds(r0, _NR), :],
            bufs.at[b, :, :, pl.ds(0, 128)],
            in_sem.at[b],
        )

    def _row_out_copy(item, b, r):
        c, i, r0 = _coords(item)
        h = i * _STRIDE + r0 + r
        return pltpu.make_async_copy(
            rows.at[b, r],
            out.at[c, h // 8, :, h % 8, :],
            out_sem.at[b],
        )

    def _assemble_rows(b, nrows, item=None):
        iota = lax.iota(jnp.int32, 16)

        @plsc.parallel_loop(0, nrows, 1)
        def _row(r):
            for v in range(64):
                w0 = 16 * v
                j = w0 // _STRIDE
                t, off = divmod(w0, 128)
                a = bufs[b, j, r, pl.ds(w0 - j * _STRIDE, 16)]
                bound = (j + 1) * _STRIDE
                if j < 8 and w0 + 16 > bound:
                    d = bound - w0
                    nxt = bufs[b, j + 1, r, pl.ds(0, 16)]
                    idx = jnp.maximum(iota - d, 0)
                    shifted = lax.gather(
                        nxt,
                        idx[:, None],
                        lax.GatherDimensionNumbers(
                            offset_dims=(),
                            collapsed_slice_dims=(0,),
                            start_index_map=(0,),
                        ),
                        (1,),
                        mode=lax.GatherScatterMode.PROMISE_IN_BOUNDS,
                    )
                    a = jnp.where(iota < d, a, shifted)
                rows[b, r, t, pl.ds(off, 16)] = a
            if item is not None:
                _row_out_copy(item, b, r).start()

    def _drain_out(b, nrows):
        pltpu.make_async_copy(
            x.at[pl.ds(0, nrows), 0, pl.ds(0, 8), :],
            rows.at[b, pl.ds(0, nrows)],
            out_sem.at[b],
        ).wait()

    _in_copy(0, 0).start()
    _in_copy(1, 1).start()

    def _pipe(t, carry):
        for b in range(2):
            item = 2 * t + b

            @pl.when(item >= 2)
            def _wait_out():
                _drain_out(b, _NR)

            _in_copy(item, b).wait()
            _assemble_rows(b, _NR, item)

            @pl.when(item + 2 < _ITEMS_PER_W)
            def _next_in():
                _in_copy(item + 2, b).start()

        return carry

    lax.fori_loop(0, _STEPS, _pipe, 0)
    _drain_out(0, _NR)
    _drain_out(1, _NR)

    @pl.when(wid < _C)
    def _():
        c = wid
        pltpu.sync_copy(
            x.at[pl.ds(8 * _GRID, _GRID), c, pl.ds(0, 16), :],
            bufs.at[0, :, pl.ds(0, 16), pl.ds(0, 128)],
        )
        _assemble_rows(0, 16)
        for r in range(16):
            h = 8 * _STRIDE + r
            pltpu.make_async_copy(
                rows.at[0, r],
                out.at[c, h // 8, :, h % 8, :],
                out_sem.at[0],
            ).start()
        _drain_out(0, 16)


_sc_kernel = functools.partial(
    pl.kernel,
    out_type=jax.ShapeDtypeStruct((_C, _H // 8, 8, 8, 128), jnp.float32),
    mesh=plsc.VectorSubcoreMesh(core_axis_name="c", subcore_axis_name="s"),
    scratch_types=[
        pltpu.VMEM((2, _GRID, _NR, _BUFP), jnp.float32),
        pltpu.VMEM((2, _NR, 8, 128), jnp.float32),
        pltpu.SemaphoreType.DMA((2,)),
        pltpu.SemaphoreType.DMA((2,)),
    ],
    compiler_params=pltpu.CompilerParams(use_tc_tiling_on_sc=False),
)(_sc_body)


def kernel(x):
    o = _sc_kernel(x)
    o = o.transpose(0, 1, 3, 2, 4)
    return o.reshape(_C, _H, _W)
